# baseline (speedup 1.0000x reference)
import functools

import jax
import jax.numpy as jnp
from jax import lax
from jax.experimental import pallas as pl
from jax.experimental.pallas import tpu as pltpu
from jax.experimental.pallas import tpu_sc as plsc

NUM_DEV = 8
R = 2048
C = 2048
NC = 2
NS = 16
NW = NC * NS
RPW = R // NW
RB = 16
NB = RPW // RB

_mesh = plsc.VectorSubcoreMesh(core_axis_name="c", subcore_axis_name="s")


@functools.partial(
    pl.kernel,
    mesh=_mesh,
    out_type=jax.ShapeDtypeStruct((NUM_DEV * R, C), jnp.float32),
    scratch_types=[
        pltpu.VMEM((16,), jnp.float32),
        pltpu.VMEM((RB, C), jnp.float32),
        pltpu.VMEM((RB, C), jnp.float32),
        pltpu.SemaphoreType.DMA,
        pltpu.SemaphoreType.DMA,
        pltpu.SemaphoreType.DMA,
        pltpu.SemaphoreType.DMA,
    ],
)
def _sc_select_broadcast(src, msk, out, mbuf, buf0, buf1,
                         gsem0, gsem1, wsem0, wsem1):
    wid = lax.axis_index("s") * NC + lax.axis_index("c")
    wbase = wid * RPW

    pltpu.sync_copy(msk, mbuf.at[pl.ds(0, NUM_DEV)])
    m = mbuf[...]
    best = m[0]
    bi = jnp.int32(0)
    for i in range(1, NUM_DEV):
        v = m[i]
        p = v > best
        bi = lax.select(p, jnp.int32(i), bi)
        best = lax.select(p, v, best)
    src_base = bi * R + wbase

    bufs = (buf0, buf1)
    gsems = (gsem0, gsem1)
    wsems = (wsem0, wsem1)

    g = pltpu.async_copy(src.at[pl.ds(src_base, RB)], bufs[0], gsems[0])
    g.wait()
    writes = [None, None]
    for i in range(NB):
        cur = i % 2
        nxt = (i + 1) % 2
        if i + 1 < NB:
            if writes[nxt] is not None:
                for h in writes[nxt]:
                    h.wait()
                writes[nxt] = None
            g = pltpu.async_copy(
                src.at[pl.ds(src_base + (i + 1) * RB, RB)],
                bufs[nxt], gsems[nxt])
        hs = []
        row = wbase + i * RB
        for d in range(NUM_DEV):
            hs.append(pltpu.async_copy(
                bufs[cur], out.at[pl.ds(d * R + row, RB)], wsems[cur]))
        writes[cur] = hs
        if i + 1 < NB:
            g.wait()
    for hl in writes:
        if hl is not None:
            for h in hl:
                h.wait()


def kernel(mesh_tensor, mask):
    src = mesh_tensor.reshape(NUM_DEV * R, C)
    out = _sc_select_broadcast(src, mask)
    return out.reshape(mesh_tensor.shape)


# speedup vs baseline: 4.9485x; 4.9485x over previous
import functools

import jax
import jax.numpy as jnp
from jax import lax
from jax.experimental import pallas as pl
from jax.experimental.pallas import tpu as pltpu
from jax.experimental.pallas import tpu_sc as plsc

NUM_DEV = 8
J = 2
RP = 2048
C = 1024
NC = 2
NS = 16
NW = NC * NS
RPW = RP * J // NW
RB = 32
NB = RPW // RB

_mesh = plsc.VectorSubcoreMesh(core_axis_name="c", subcore_axis_name="s")


@functools.partial(
    pl.kernel,
    mesh=_mesh,
    out_type=jax.ShapeDtypeStruct((NUM_DEV, J, RP, C), jnp.float32),
    scratch_types=[
        pltpu.VMEM((16,), jnp.float32),
        pltpu.VMEM((RB, C), jnp.float32),
        pltpu.VMEM((RB, C), jnp.float32),
        pltpu.SemaphoreType.DMA,
        pltpu.SemaphoreType.DMA,
        pltpu.SemaphoreType.DMA,
        pltpu.SemaphoreType.DMA,
    ],
)
def _sc_select_broadcast(src, msk, out, mbuf, buf0, buf1,
                         gsem0, gsem1, wsem0, wsem1):
    wid = lax.axis_index("s") * NC + lax.axis_index("c")
    j = wid % J
    rbase = (wid // J) * RPW

    pltpu.sync_copy(msk, mbuf.at[pl.ds(0, NUM_DEV)])
    m = mbuf[...]
    best = m[0]
    bi = jnp.int32(0)
    for i in range(1, NUM_DEV):
        v = m[i]
        p = v > best
        bi = lax.select(p, jnp.int32(i), bi)
        best = lax.select(p, v, best)

    bufs = (buf0, buf1)
    gsems = (gsem0, gsem1)
    wsems = (wsem0, wsem1)

    g = pltpu.async_copy(src.at[bi, j, pl.ds(rbase, RB)], bufs[0], gsems[0])
    g.wait()
    writes = [None, None]
    for i in range(NB):
        cur = i % 2
        nxt = (i + 1) % 2
        if i + 1 < NB:
            if writes[nxt] is not None:
                for h in writes[nxt]:
                    h.wait()
                writes[nxt] = None
            g = pltpu.async_copy(
                src.at[bi, j, pl.ds(rbase + (i + 1) * RB, RB)],
                bufs[nxt], gsems[nxt])
        hs = []
        row = rbase + i * RB
        for d in range(NUM_DEV):
            hs.append(pltpu.async_copy(
                bufs[cur], out.at[d, j, pl.ds(row, RB)], wsems[cur]))
        writes[cur] = hs
        if i + 1 < NB:
            g.wait()
    for hl in writes:
        if hl is not None:
            for h in hl:
                h.wait()


def kernel(mesh_tensor, mask):
    return _sc_select_broadcast(mesh_tensor, mask)


# speedup vs baseline: 5.0750x; 1.0256x over previous
import functools

import jax
import jax.numpy as jnp
from jax import lax
from jax.experimental import pallas as pl
from jax.experimental.pallas import tpu as pltpu
from jax.experimental.pallas import tpu_sc as plsc

NUM_DEV = 8
J = 2
RP = 2048
C = 1024
NC = 2
NS = 16
NW = NC * NS
RPW = RP * J // NW
RB = 64
NB = RPW // RB

_mesh = plsc.VectorSubcoreMesh(core_axis_name="c", subcore_axis_name="s")


@functools.partial(
    pl.kernel,
    mesh=_mesh,
    out_type=jax.ShapeDtypeStruct((NUM_DEV, J, RP, C), jnp.float32),
    scratch_types=[
        pltpu.VMEM((16,), jnp.float32),
        pltpu.VMEM((RB, C), jnp.float32),
        pltpu.SemaphoreType.DMA,
        pltpu.SemaphoreType.DMA,
    ],
)
def _sc_select_broadcast(src, msk, out, mbuf, buf0, gsem0, wsem0):
    wid = lax.axis_index("s") * NC + lax.axis_index("c")
    j = wid % J
    rbase = (wid // J) * RPW

    pltpu.sync_copy(msk, mbuf.at[pl.ds(0, NUM_DEV)])
    m = mbuf[...]
    best = m[0]
    bi = jnp.int32(0)
    for i in range(1, NUM_DEV):
        v = m[i]
        p = v > best
        bi = lax.select(p, jnp.int32(i), bi)
        best = lax.select(p, v, best)

    for i in range(NB):
        row = rbase + i * RB
        pltpu.async_copy(src.at[bi, j, pl.ds(row, RB)], buf0, gsem0).wait()
        hs = []
        for d in range(NUM_DEV):
            hs.append(pltpu.async_copy(
                buf0, out.at[d, j, pl.ds(row, RB)], wsem0))
        for h in hs:
            h.wait()


def kernel(mesh_tensor, mask):
    return _sc_select_broadcast(mesh_tensor, mask)


# speedup vs baseline: 5.1074x; 1.0064x over previous
import functools

import jax
import jax.numpy as jnp
from jax import lax
from jax.experimental import pallas as pl
from jax.experimental.pallas import tpu as pltpu
from jax.experimental.pallas import tpu_sc as plsc

NUM_DEV = 8
J = 2
RP = 2048
C = 1024
NC = 2
NS = 16
NW = NC * NS
RPW = RP * J // NW
RB0 = 64
RB1 = 56

_mesh = plsc.VectorSubcoreMesh(core_axis_name="c", subcore_axis_name="s")


@functools.partial(
    pl.kernel,
    mesh=_mesh,
    out_type=jax.ShapeDtypeStruct((NUM_DEV, J, RP, C), jnp.float32),
    scratch_types=[
        pltpu.VMEM((16,), jnp.float32),
        pltpu.VMEM((RB0, C), jnp.float32),
        pltpu.VMEM((RB1, C), jnp.float32),
        pltpu.SemaphoreType.DMA,
        pltpu.SemaphoreType.DMA,
        pltpu.SemaphoreType.DMA,
        pltpu.SemaphoreType.DMA,
    ],
)
def _sc_select_broadcast(src, msk, out, mbuf, buf0, buf1,
                         gsem0, gsem1, wsem0, wsem1):
    wid = lax.axis_index("s") * NC + lax.axis_index("c")
    j = wid % J
    rbase = (wid // J) * RPW

    pltpu.sync_copy(msk, mbuf.at[pl.ds(0, NUM_DEV)])
    m = mbuf[...]
    best = m[0]
    bi = jnp.int32(0)
    for i in range(1, NUM_DEV):
        v = m[i]
        p = v > best
        bi = lax.select(p, jnp.int32(i), bi)
        best = lax.select(p, v, best)

    r0, r1, r2 = rbase, rbase + RB0, rbase + RB0 + RB1
    btail = buf0.at[pl.ds(0, RPW - RB0 - RB1)]

    pltpu.async_copy(src.at[bi, j, pl.ds(r0, RB0)], buf0, gsem0).wait()
    g1 = pltpu.async_copy(src.at[bi, j, pl.ds(r1, RB1)], buf1, gsem1)
    w0 = [pltpu.async_copy(buf0, out.at[d, j, pl.ds(r0, RB0)], wsem0)
          for d in range(NUM_DEV)]
    for h in w0:
        h.wait()
    g2 = pltpu.async_copy(
        src.at[bi, j, pl.ds(r2, RPW - RB0 - RB1)], btail, gsem0)
    g1.wait()
    w1 = [pltpu.async_copy(buf1, out.at[d, j, pl.ds(r1, RB1)], wsem1)
          for d in range(NUM_DEV)]
    for h in w1:
        h.wait()
    g2.wait()
    w2 = [pltpu.async_copy(btail, out.at[d, j, pl.ds(r2, RPW - RB0 - RB1)],
                           wsem0)
          for d in range(NUM_DEV)]
    for h in w2:
        h.wait()


def kernel(mesh_tensor, mask):
    return _sc_select_broadcast(mesh_tensor, mask)


# speedup vs baseline: 5.1392x; 1.0062x over previous
import functools

import jax
import jax.numpy as jnp
from jax import lax
from jax.experimental import pallas as pl
from jax.experimental.pallas import tpu as pltpu
from jax.experimental.pallas import tpu_sc as plsc

NUM_DEV = 8
J = 2
RP = 2048
C = 1024
NC = 2
NS = 16
NW = NC * NS
RPW = RP * J // NW
RB0 = 8
RB1 = 64
RB2 = 56

_mesh = plsc.VectorSubcoreMesh(core_axis_name="c", subcore_axis_name="s")


@functools.partial(
    pl.kernel,
    mesh=_mesh,
    out_type=jax.ShapeDtypeStruct((NUM_DEV, J, RP, C), jnp.float32),
    scratch_types=[
        pltpu.VMEM((16,), jnp.float32),
        pltpu.VMEM((RB2, C), jnp.float32),
        pltpu.VMEM((RB1, C), jnp.float32),
        pltpu.SemaphoreType.DMA,
        pltpu.SemaphoreType.DMA,
        pltpu.SemaphoreType.DMA,
        pltpu.SemaphoreType.DMA,
    ],
)
def _sc_select_broadcast(src, msk, out, mbuf, buf0, buf1,
                         gsem0, gsem1, wsem0, wsem1):
    wid = lax.axis_index("s") * NC + lax.axis_index("c")
    j = wid % J
    rbase = (wid // J) * RPW

    pltpu.sync_copy(msk, mbuf.at[pl.ds(0, NUM_DEV)])
    m = mbuf[...]
    best = m[0]
    bi = jnp.int32(0)
    for i in range(1, NUM_DEV):
        v = m[i]
        p = v > best
        bi = lax.select(p, jnp.int32(i), bi)
        best = lax.select(p, v, best)

    r0, r1, r2 = rbase, rbase + RB0, rbase + RB0 + RB1
    bramp = buf0.at[pl.ds(0, RB0)]

    g0 = pltpu.async_copy(src.at[bi, j, pl.ds(r0, RB0)], bramp, gsem0)
    g1 = pltpu.async_copy(src.at[bi, j, pl.ds(r1, RB1)], buf1, gsem1)
    g0.wait()
    w0 = [pltpu.async_copy(bramp, out.at[d, j, pl.ds(r0, RB0)], wsem0)
          for d in range(NUM_DEV)]
    for h in w0:
        h.wait()
    g2 = pltpu.async_copy(src.at[bi, j, pl.ds(r2, RB2)], buf0, gsem0)
    g1.wait()
    w1 = [pltpu.async_copy(buf1, out.at[d, j, pl.ds(r1, RB1)], wsem1)
          for d in range(NUM_DEV)]
    for h in w1:
        h.wait()
    g2.wait()
    w2 = [pltpu.async_copy(buf0, out.at[d, j, pl.ds(r2, RB2)], wsem0)
          for d in range(NUM_DEV)]
    for h in w2:
        h.wait()


def kernel(mesh_tensor, mask):
    return _sc_select_broadcast(mesh_tensor, mask)
